# baseline (speedup 1.0000x reference)
import functools

import jax
import jax.numpy as jnp
from jax import lax
from jax.experimental import pallas as pl
from jax.experimental.pallas import tpu as pltpu
from jax.experimental.pallas import tpu_sc as plsc

NUM_EMB = 8192
EMB_DIM = 1024

NC = 2
NS = 16
NW = NC * NS

B_TOTAL = 4 * 8192
R = B_TOTAL // NW
CHUNK = 8
NBUF = 8
NSH = 4
NCHUNK = R // CHUNK
K_OUTER = NCHUNK // NBUF


def _emb_body(idx_hbm, table_hbm, out_hbm, idx_v, buf_v, shr, gsem, xsem, ssem):
    sid = lax.axis_index("s")
    wid = sid * NC + lax.axis_index("c")
    base = wid * R

    pltpu.sync_copy(idx_hbm.at[pl.ds(base, R)], idx_v)

    def gather_start(i, b):
        pltpu.async_copy(
            table_hbm.at[idx_v.at[pl.ds(i * CHUNK, CHUNK)]],
            buf_v.at[b],
            gsem.at[b],
        )

    def gather_wait(b):
        pltpu.make_async_copy(
            table_hbm.at[idx_v.at[pl.ds(0, CHUNK)]], buf_v.at[b], gsem.at[b]
        ).wait()

    def cross_start(b, s):
        pltpu.async_copy(buf_v.at[b], shr.at[sid, s], xsem.at[b])

    def cross_wait(b, s):
        pltpu.make_async_copy(buf_v.at[b], shr.at[sid, s], xsem.at[b]).wait()

    def store_start(i, s):
        pltpu.async_copy(
            shr.at[sid, s], out_hbm.at[pl.ds(base + i * CHUNK, CHUNK)], ssem.at[s]
        )

    def store_wait(s):
        pltpu.make_async_copy(
            shr.at[sid, s], out_hbm.at[pl.ds(base, CHUNK)], ssem.at[s]
        ).wait()

    for b in range(4):
        gather_start(b, b)

    def outer(k, carry):
        for u in range(NBUF):
            i = k * NBUF + u
            gather_wait(u)
            if u >= NSH:
                store_wait(u - NSH)
            else:
                @pl.when(k > 0)
                def _():
                    store_wait(u)
            cross_start(u, u % NSH)
            if u >= 2:
                cross_wait(u - 2, (u - 2) % NSH)
                store_start(i - 2, (u - 2) % NSH)
            else:
                @pl.when(k > 0)
                def _():
                    cross_wait((u - 2) % NBUF, (u - 2) % NSH)
                    store_start(i - 2, (u - 2) % NSH)
            if u < 4:
                gather_start(i + 4, u + 4)
            else:
                @pl.when(k < K_OUTER - 1)
                def _():
                    gather_start(i + 4, (u + 4) % NBUF)
        return carry

    lax.fori_loop(0, K_OUTER, outer, 0)

    N = NCHUNK
    for j in (N - 2, N - 1):
        cross_wait(j % NBUF, j % NSH)
        store_start(j, j % NSH)
    for j in range(N - NSH, N):
        store_wait(j % NSH)


@functools.partial(jax.jit, static_argnames=())
def _lookup(idx_flat, weights):
    mesh = plsc.VectorSubcoreMesh(core_axis_name="c", subcore_axis_name="s")
    return pl.kernel(
        _emb_body,
        out_type=jax.ShapeDtypeStruct((B_TOTAL, EMB_DIM), jnp.float32),
        mesh=mesh,
        scratch_types=[
            pltpu.VMEM((R,), jnp.int32),
            pltpu.VMEM((NBUF, CHUNK, EMB_DIM), jnp.float32),
            pltpu.VMEM_SHARED((NS, NSH, CHUNK, EMB_DIM), jnp.float32),
            pltpu.SemaphoreType.DMA((NBUF,)),
            pltpu.SemaphoreType.DMA((NBUF,)),
            pltpu.SemaphoreType.DMA((NSH,)),
        ],
    )(idx_flat, weights)


def kernel(position_ids, weights):
    batch, length = position_ids.shape
    out = _lookup(position_ids.reshape(-1), weights)
    return out.reshape(batch, length, EMB_DIM)
